# baseline (speedup 1.0000x reference)
import functools

import jax
import jax.numpy as jnp
from jax import lax
from jax.experimental import pallas as pl
from jax.experimental.pallas import tpu as pltpu
from jax.experimental.pallas import tpu_sc as plsc

NC = 2
NS = 16
NW = NC * NS
CH = 128
C0_FRAC = 0.25
LANES = 16


def _fill(ref, val):
    rows, cols = ref.shape
    nvec = cols // LANES
    v = jnp.full((LANES,), val, jnp.float32)

    def body(i, carry):
        r = i // nvec
        col = (i % nvec) * LANES
        ref[r, pl.ds(col, LANES)] = v
        return carry

    lax.fori_loop(0, rows * nvec, body, 0)


def _mesh():
    return plsc.VectorSubcoreMesh(core_axis_name="c", subcore_axis_name="s",
                                  num_cores=NC, num_subcores=NS)


def _make_sc_feat_sum(n_acc, ng0, ng1, d):
    rows_per_tile = n_acc // NS
    n_groups = max(ng0, ng1)

    @functools.partial(
        pl.kernel,
        out_type=jax.ShapeDtypeStruct((NC, n_acc, d), jnp.float32),
        mesh=_mesh(),
        scratch_types=[
            pltpu.VMEM((2, 2, CH), jnp.int32),
            pltpu.VMEM((2, 2, CH), jnp.int32),
            pltpu.VMEM((CH, d), jnp.float32),
            pltpu.VMEM((CH, d), jnp.float32),
            pltpu.VMEM_SHARED((n_acc, d), jnp.float32),
            pltpu.SemaphoreType.DMA,
            pltpu.SemaphoreType.DMA,
            pltpu.SemaphoreType.DMA,
            pltpu.SemaphoreType.DMA,
            pltpu.SemaphoreType.DMA,
        ],
    )
    def sc_feat_sum(feat_hbm, srcs_hbm, dsts_hbm, acc_out,
                    src_v, dst_v, rbuf0, rbuf1, acc_sp,
                    semg0, semg1, sems0, sems1, sem_idx):
        c = lax.axis_index("c")
        s = lax.axis_index("s")
        w = c * NS + s
        ngc = lax.select(c == 0, jnp.int32(ng0), jnp.int32(ng1))
        base = pl.multiple_of(s * rows_per_tile, 8)

        _fill(rbuf0, 0.0)
        n_full = rows_per_tile // CH
        rem = rows_per_tile - n_full * CH
        for k in range(n_full):
            pltpu.sync_copy(rbuf0, acc_sp.at[pl.ds(base + k * CH, CH)])
        if rem:
            pltpu.sync_copy(rbuf0.at[pl.ds(0, rem)],
                            acc_sp.at[pl.ds(base + n_full * CH, rem)])

        plsc.subcore_barrier()

        H = CH // 2

        def gather2(slot, k, rbuf, sem):
            pltpu.async_copy(feat_hbm.at[src_v.at[slot, k, pl.ds(0, H)]],
                             rbuf.at[pl.ds(0, H)], sem)
            pltpu.async_copy(feat_hbm.at[src_v.at[slot, k, pl.ds(H, H)]],
                             rbuf.at[pl.ds(H, H)], sem)

        def gather2_wait(slot, k, rbuf, sem):
            pltpu.make_async_copy(feat_hbm.at[src_v.at[slot, k, pl.ds(0, H)]],
                                  rbuf.at[pl.ds(0, H)], sem).wait()
            pltpu.make_async_copy(feat_hbm.at[src_v.at[slot, k, pl.ds(H, H)]],
                                  rbuf.at[pl.ds(H, H)], sem).wait()

        pltpu.sync_copy(srcs_hbm.at[w * n_groups], src_v.at[0])
        pltpu.sync_copy(dsts_hbm.at[w * n_groups], dst_v.at[0])
        gather2(0, 0, rbuf0, semg0)
        gather2(0, 1, rbuf1, semg1)

        def body(g, carry):
            slot = g % 2
            nxt = (g + 1) % 2
            have_next = g + 1 < ngc

            @pl.when(have_next)
            def _():
                pltpu.async_copy(srcs_hbm.at[w * n_groups + g + 1],
                                 src_v.at[nxt], sem_idx)
                pltpu.async_copy(dsts_hbm.at[w * n_groups + g + 1],
                                 dst_v.at[nxt], sem_idx)

            gather2_wait(slot, 0, rbuf0, semg0)
            pltpu.async_copy(rbuf0, acc_sp.at[dst_v.at[slot, 0]], sems0,
                             add=True)
            gather2_wait(slot, 1, rbuf1, semg1)
            pltpu.async_copy(rbuf1, acc_sp.at[dst_v.at[slot, 1]], sems1,
                             add=True)

            @pl.when(have_next)
            def _():
                pltpu.make_async_copy(srcs_hbm.at[w * n_groups + g + 1],
                                      src_v.at[nxt], sem_idx).wait()
                pltpu.make_async_copy(dsts_hbm.at[w * n_groups + g + 1],
                                      dst_v.at[nxt], sem_idx).wait()
                pltpu.make_async_copy(rbuf0, acc_sp.at[dst_v.at[slot, 0]],
                                      sems0).wait()
                gather2(nxt, 0, rbuf0, semg0)
                pltpu.make_async_copy(rbuf1, acc_sp.at[dst_v.at[slot, 1]],
                                      sems1).wait()
                gather2(nxt, 1, rbuf1, semg1)

            @pl.when(jnp.logical_not(have_next))
            def _():
                pltpu.make_async_copy(rbuf0, acc_sp.at[dst_v.at[slot, 0]],
                                      sems0).wait()
                pltpu.make_async_copy(rbuf1, acc_sp.at[dst_v.at[slot, 1]],
                                      sems1).wait()
            return carry

        lax.fori_loop(0, ngc, body, 0)

        plsc.subcore_barrier()

        pltpu.sync_copy(acc_sp.at[pl.ds(base, rows_per_tile)],
                        acc_out.at[c, pl.ds(base, rows_per_tile)])

    return sc_feat_sum


def _make_sc_degree(n_deg, ept):
    cols_per_tile = n_deg // NS

    @functools.partial(
        pl.kernel,
        out_type=jax.ShapeDtypeStruct((NC, n_deg), jnp.float32),
        mesh=_mesh(),
        compiler_params=pltpu.CompilerParams(needs_layout_passes=False),
        scratch_types=[
            pltpu.VMEM((ept,), jnp.int32),
            pltpu.VMEM((n_deg,), jnp.float32),
            pltpu.VMEM((NS, n_deg // NS), jnp.float32),
            pltpu.VMEM((n_deg // NS,), jnp.float32),
            pltpu.VMEM_SHARED((NS, n_deg), jnp.float32),
        ],
    )
    def sc_degree(dsts_hbm, deg_out, dst_v, hist_v, red_v, out_v, stage_sp):
        c = lax.axis_index("c")
        s = lax.axis_index("s")
        w = c * NS + s
        zero16 = jnp.zeros((LANES,), jnp.float32)
        one16 = jnp.ones((LANES,), jnp.float32)

        def zbody(i, carry):
            hist_v[pl.ds(i * LANES, LANES)] = zero16
            return carry

        lax.fori_loop(0, n_deg // LANES, zbody, 0)

        pltpu.sync_copy(dsts_hbm.at[w], dst_v)

        def hbody(i, carry):
            idx16 = dst_v[pl.ds(i * LANES, LANES)]
            plsc.addupdate_scatter(hist_v, [idx16], one16)
            return carry

        lax.fori_loop(0, ept // LANES, hbody, 0)

        pltpu.sync_copy(hist_v, stage_sp.at[s])
        plsc.subcore_barrier()

        base = pl.multiple_of(s * cols_per_tile, 8)
        pltpu.sync_copy(stage_sp.at[:, pl.ds(base, cols_per_tile)], red_v)

        def rbody(j, carry):
            v = red_v[0, pl.ds(j * LANES, LANES)]
            for t in range(1, NS):
                v = v + red_v[t, pl.ds(j * LANES, LANES)]
            out_v[pl.ds(j * LANES, LANES)] = v
            return carry

        lax.fori_loop(0, cols_per_tile // LANES, rbody, 0)

        pltpu.sync_copy(out_v, deg_out.at[c, pl.ds(base, cols_per_tile)])

    return sc_degree


def _tc_body(eps_ref, acc_ref, deg_ref, feat_ref, w1_ref, b1_ref, w2_ref,
             b2_ref, out_ref):
    summed = acc_ref[0] + acc_ref[1]
    deg = (deg_ref[0] + deg_ref[1])[:, None]
    neigh = summed / jnp.maximum(deg, 1.0)
    rst = (1.0 + eps_ref[0]) * feat_ref[...] + neigh
    h = jnp.dot(rst, w1_ref[...], preferred_element_type=jnp.float32)
    h = jnp.maximum(h + b1_ref[...], 0.0)
    o = jnp.dot(h, w2_ref[...], preferred_element_type=jnp.float32)
    out_ref[...] = jnp.maximum(o + b2_ref[...], 0.0)


def kernel(feat, edge_index, eps, W1, b1, W2, b2):
    n, d = feat.shape
    e = edge_index.shape[1]
    d_hid = W1.shape[1]
    d_out = W2.shape[1]

    n_acc = -(-(n + 1) // (NS * 8)) * (NS * 8)

    src = edge_index[0].astype(jnp.int32)
    dst = edge_index[1].astype(jnp.int32)

    def pack_half(a, fill, ng_pad):
        e_c = a.shape[0]
        ept_c = -(-e_c // (NS * 2 * CH)) * (2 * CH)
        ng_c = ept_c // (2 * CH)
        a = jnp.concatenate(
            [a, jnp.full((NS * ept_c - e_c,), fill, jnp.int32)])
        a = a.reshape(NS, ng_c, 2, CH)
        return jnp.pad(a, ((0, 0), (0, ng_pad - ng_c), (0, 0), (0, 0)),
                       constant_values=fill), ng_c

    e0 = int(e * C0_FRAC) & ~255
    ng0 = -(-max(e0, 1) // (NS * 2 * CH))
    ng1 = -(-max(e - e0, 1) // (NS * 2 * CH))
    ng_max = max(ng0, ng1)
    src_u0, _ = pack_half(src[:e0], 0, ng_max)
    src_u1, _ = pack_half(src[e0:] + n, n, ng_max)
    dst_u0, _ = pack_half(dst[:e0], n, ng_max)
    dst_u1, _ = pack_half(dst[e0:], n, ng_max)
    src_u = jnp.concatenate([src_u0, src_u1]).reshape(NW * ng_max, 2, CH)
    dst_u = jnp.concatenate([dst_u0, dst_u1]).reshape(NW * ng_max, 2, CH)

    ept = -(-e // (NW * 2 * CH)) * (2 * CH)
    e_pad = ept * NW
    n_deg = -(-(n + 1) // (NS * LANES)) * (NS * LANES)
    dst_d = jnp.concatenate(
        [dst, jnp.full((e_pad - e,), n, jnp.int32)]).reshape(NW, ept)

    feat2 = jnp.concatenate([feat, feat], axis=0)

    acc = _make_sc_feat_sum(n_acc, ng0, ng1, d)(feat2, src_u, dst_u)
    deg = _make_sc_degree(n_deg, ept)(dst_d)

    blk = 512
    grid = -(-n // blk)
    out = pl.pallas_call(
        _tc_body,
        grid=(grid,),
        in_specs=[
            pl.BlockSpec(memory_space=pltpu.SMEM),
            pl.BlockSpec((NC, blk, d), lambda i: (0, i, 0)),
            pl.BlockSpec((NC, blk), lambda i: (0, i)),
            pl.BlockSpec((blk, d), lambda i: (i, 0)),
            pl.BlockSpec((d, d_hid), lambda i: (0, 0)),
            pl.BlockSpec((1, d_hid), lambda i: (0, 0)),
            pl.BlockSpec((d_hid, d_out), lambda i: (0, 0)),
            pl.BlockSpec((1, d_out), lambda i: (0, 0)),
        ],
        out_specs=pl.BlockSpec((blk, d_out), lambda i: (i, 0)),
        out_shape=jax.ShapeDtypeStruct((n, d_out), jnp.float32),
    )(eps, acc, deg, feat, W1, b1.reshape(1, d_hid), W2, b2.reshape(1, d_out))
    return out


# speedup vs baseline: 1.0471x; 1.0471x over previous
import functools

import jax
import jax.numpy as jnp
from jax import lax
from jax.experimental import pallas as pl
from jax.experimental.pallas import tpu as pltpu
from jax.experimental.pallas import tpu_sc as plsc

NC = 2
NS = 16
NW = NC * NS
CH = 128
C0_FRAC = 0.22
LANES = 16


def _fill(ref, val):
    rows, cols = ref.shape
    nvec = cols // LANES
    v = jnp.full((LANES,), val, jnp.float32)

    def body(i, carry):
        r = i // nvec
        col = (i % nvec) * LANES
        ref[r, pl.ds(col, LANES)] = v
        return carry

    lax.fori_loop(0, rows * nvec, body, 0)


def _mesh():
    return plsc.VectorSubcoreMesh(core_axis_name="c", subcore_axis_name="s",
                                  num_cores=NC, num_subcores=NS)


def _make_sc_feat_sum(n_acc, ng0, ng1, d):
    rows_per_tile = n_acc // NS
    n_groups = max(ng0, ng1)

    @functools.partial(
        pl.kernel,
        out_type=jax.ShapeDtypeStruct((NC, n_acc, d), jnp.float32),
        mesh=_mesh(),
        scratch_types=[
            pltpu.VMEM((2, 2, CH), jnp.int32),
            pltpu.VMEM((2, 2, CH), jnp.int32),
            pltpu.VMEM((CH, d), jnp.float32),
            pltpu.VMEM((CH, d), jnp.float32),
            pltpu.VMEM_SHARED((n_acc, d), jnp.float32),
            pltpu.SemaphoreType.DMA,
            pltpu.SemaphoreType.DMA,
            pltpu.SemaphoreType.DMA,
            pltpu.SemaphoreType.DMA,
            pltpu.SemaphoreType.DMA,
        ],
    )
    def sc_feat_sum(feat_hbm, srcs_hbm, dsts_hbm, acc_out,
                    src_v, dst_v, rbuf0, rbuf1, acc_sp,
                    semg0, semg1, sems0, sems1, sem_idx):
        c = lax.axis_index("c")
        s = lax.axis_index("s")
        w = c * NS + s
        ngc = lax.select(c == 0, jnp.int32(ng0), jnp.int32(ng1))
        base = pl.multiple_of(s * rows_per_tile, 8)

        _fill(rbuf0, 0.0)
        n_full = rows_per_tile // CH
        rem = rows_per_tile - n_full * CH
        for k in range(n_full):
            pltpu.sync_copy(rbuf0, acc_sp.at[pl.ds(base + k * CH, CH)])
        if rem:
            pltpu.sync_copy(rbuf0.at[pl.ds(0, rem)],
                            acc_sp.at[pl.ds(base + n_full * CH, rem)])

        plsc.subcore_barrier()

        H = CH // 2

        def gather2(slot, k, rbuf, sem):
            pltpu.async_copy(feat_hbm.at[src_v.at[slot, k, pl.ds(0, H)]],
                             rbuf.at[pl.ds(0, H)], sem)
            pltpu.async_copy(feat_hbm.at[src_v.at[slot, k, pl.ds(H, H)]],
                             rbuf.at[pl.ds(H, H)], sem)

        def gather2_wait(slot, k, rbuf, sem):
            pltpu.make_async_copy(feat_hbm.at[src_v.at[slot, k, pl.ds(0, H)]],
                                  rbuf.at[pl.ds(0, H)], sem).wait()
            pltpu.make_async_copy(feat_hbm.at[src_v.at[slot, k, pl.ds(H, H)]],
                                  rbuf.at[pl.ds(H, H)], sem).wait()

        pltpu.sync_copy(srcs_hbm.at[w * n_groups], src_v.at[0])
        pltpu.sync_copy(dsts_hbm.at[w * n_groups], dst_v.at[0])
        gather2(0, 0, rbuf0, semg0)
        gather2(0, 1, rbuf1, semg1)

        def body(g, carry):
            slot = g % 2
            nxt = (g + 1) % 2
            have_next = g + 1 < ngc

            @pl.when(have_next)
            def _():
                pltpu.async_copy(srcs_hbm.at[w * n_groups + g + 1],
                                 src_v.at[nxt], sem_idx)
                pltpu.async_copy(dsts_hbm.at[w * n_groups + g + 1],
                                 dst_v.at[nxt], sem_idx)

            gather2_wait(slot, 0, rbuf0, semg0)
            pltpu.async_copy(rbuf0, acc_sp.at[dst_v.at[slot, 0]], sems0,
                             add=True)
            gather2_wait(slot, 1, rbuf1, semg1)
            pltpu.async_copy(rbuf1, acc_sp.at[dst_v.at[slot, 1]], sems1,
                             add=True)

            @pl.when(have_next)
            def _():
                pltpu.make_async_copy(srcs_hbm.at[w * n_groups + g + 1],
                                      src_v.at[nxt], sem_idx).wait()
                pltpu.make_async_copy(dsts_hbm.at[w * n_groups + g + 1],
                                      dst_v.at[nxt], sem_idx).wait()
                pltpu.make_async_copy(rbuf0, acc_sp.at[dst_v.at[slot, 0]],
                                      sems0).wait()
                gather2(nxt, 0, rbuf0, semg0)
                pltpu.make_async_copy(rbuf1, acc_sp.at[dst_v.at[slot, 1]],
                                      sems1).wait()
                gather2(nxt, 1, rbuf1, semg1)

            @pl.when(jnp.logical_not(have_next))
            def _():
                pltpu.make_async_copy(rbuf0, acc_sp.at[dst_v.at[slot, 0]],
                                      sems0).wait()
                pltpu.make_async_copy(rbuf1, acc_sp.at[dst_v.at[slot, 1]],
                                      sems1).wait()
            return carry

        lax.fori_loop(0, ngc, body, 0)

        plsc.subcore_barrier()

        pltpu.sync_copy(acc_sp.at[pl.ds(base, rows_per_tile)],
                        acc_out.at[c, pl.ds(base, rows_per_tile)])

    return sc_feat_sum


def _make_sc_degree(n_deg, ept):
    cols_per_tile = n_deg // NS

    @functools.partial(
        pl.kernel,
        out_type=jax.ShapeDtypeStruct((NC, n_deg), jnp.float32),
        mesh=_mesh(),
        compiler_params=pltpu.CompilerParams(needs_layout_passes=False),
        scratch_types=[
            pltpu.VMEM((ept,), jnp.int32),
            pltpu.VMEM((n_deg,), jnp.float32),
            pltpu.VMEM((NS, n_deg // NS), jnp.float32),
            pltpu.VMEM((n_deg // NS,), jnp.float32),
            pltpu.VMEM_SHARED((NS, n_deg), jnp.float32),
        ],
    )
    def sc_degree(dsts_hbm, deg_out, dst_v, hist_v, red_v, out_v, stage_sp):
        c = lax.axis_index("c")
        s = lax.axis_index("s")
        w = c * NS + s
        zero16 = jnp.zeros((LANES,), jnp.float32)
        one16 = jnp.ones((LANES,), jnp.float32)

        def zbody(i, carry):
            hist_v[pl.ds(i * LANES, LANES)] = zero16
            return carry

        lax.fori_loop(0, n_deg // LANES, zbody, 0)

        pltpu.sync_copy(dsts_hbm.at[w], dst_v)

        def hbody(i, carry):
            idx16 = dst_v[pl.ds(i * LANES, LANES)]
            plsc.addupdate_scatter(hist_v, [idx16], one16)
            return carry

        lax.fori_loop(0, ept // LANES, hbody, 0)

        pltpu.sync_copy(hist_v, stage_sp.at[s])
        plsc.subcore_barrier()

        base = pl.multiple_of(s * cols_per_tile, 8)
        pltpu.sync_copy(stage_sp.at[:, pl.ds(base, cols_per_tile)], red_v)

        def rbody(j, carry):
            v = red_v[0, pl.ds(j * LANES, LANES)]
            for t in range(1, NS):
                v = v + red_v[t, pl.ds(j * LANES, LANES)]
            out_v[pl.ds(j * LANES, LANES)] = v
            return carry

        lax.fori_loop(0, cols_per_tile // LANES, rbody, 0)

        pltpu.sync_copy(out_v, deg_out.at[c, pl.ds(base, cols_per_tile)])

    return sc_degree


def _tc_body(eps_ref, acc_ref, deg_ref, feat_ref, w1_ref, b1_ref, w2_ref,
             b2_ref, out_ref):
    summed = acc_ref[0] + acc_ref[1]
    deg = (deg_ref[0] + deg_ref[1])[:, None]
    neigh = summed / jnp.maximum(deg, 1.0)
    rst = (1.0 + eps_ref[0]) * feat_ref[...] + neigh
    h = jnp.dot(rst, w1_ref[...], preferred_element_type=jnp.float32)
    h = jnp.maximum(h + b1_ref[...], 0.0)
    o = jnp.dot(h, w2_ref[...], preferred_element_type=jnp.float32)
    out_ref[...] = jnp.maximum(o + b2_ref[...], 0.0)


def kernel(feat, edge_index, eps, W1, b1, W2, b2):
    n, d = feat.shape
    e = edge_index.shape[1]
    d_hid = W1.shape[1]
    d_out = W2.shape[1]

    n_acc = -(-(n + 1) // (NS * 8)) * (NS * 8)

    src = edge_index[0].astype(jnp.int32)
    dst = edge_index[1].astype(jnp.int32)

    def pack_half(a, fill, ng_pad):
        e_c = a.shape[0]
        ept_c = -(-e_c // (NS * 2 * CH)) * (2 * CH)
        ng_c = ept_c // (2 * CH)
        a = jnp.concatenate(
            [a, jnp.full((NS * ept_c - e_c,), fill, jnp.int32)])
        a = a.reshape(NS, ng_c, 2, CH)
        return jnp.pad(a, ((0, 0), (0, ng_pad - ng_c), (0, 0), (0, 0)),
                       constant_values=fill), ng_c

    e0 = int(e * C0_FRAC) & ~255
    ng0 = -(-max(e0, 1) // (NS * 2 * CH))
    ng1 = -(-max(e - e0, 1) // (NS * 2 * CH))
    ng_max = max(ng0, ng1)
    src_u0, _ = pack_half(src[:e0], 0, ng_max)
    src_u1, _ = pack_half(src[e0:] + n, n, ng_max)
    dst_u0, _ = pack_half(dst[:e0], n, ng_max)
    dst_u1, _ = pack_half(dst[e0:], n, ng_max)
    src_u = jnp.concatenate([src_u0, src_u1]).reshape(NW * ng_max, 2, CH)
    dst_u = jnp.concatenate([dst_u0, dst_u1]).reshape(NW * ng_max, 2, CH)

    ept = -(-e // (NW * 2 * CH)) * (2 * CH)
    e_pad = ept * NW
    n_deg = -(-(n + 1) // (NS * LANES)) * (NS * LANES)
    dst_d = jnp.concatenate(
        [dst, jnp.full((e_pad - e,), n, jnp.int32)]).reshape(NW, ept)

    feat2 = jnp.concatenate([feat, feat], axis=0)

    acc = _make_sc_feat_sum(n_acc, ng0, ng1, d)(feat2, src_u, dst_u)
    deg = _make_sc_degree(n_deg, ept)(dst_d)

    blk = 512
    grid = -(-n // blk)
    out = pl.pallas_call(
        _tc_body,
        grid=(grid,),
        in_specs=[
            pl.BlockSpec(memory_space=pltpu.SMEM),
            pl.BlockSpec((NC, blk, d), lambda i: (0, i, 0)),
            pl.BlockSpec((NC, blk), lambda i: (0, i)),
            pl.BlockSpec((blk, d), lambda i: (i, 0)),
            pl.BlockSpec((d, d_hid), lambda i: (0, 0)),
            pl.BlockSpec((1, d_hid), lambda i: (0, 0)),
            pl.BlockSpec((d_hid, d_out), lambda i: (0, 0)),
            pl.BlockSpec((1, d_out), lambda i: (0, 0)),
        ],
        out_specs=pl.BlockSpec((blk, d_out), lambda i: (i, 0)),
        out_shape=jax.ShapeDtypeStruct((n, d_out), jnp.float32),
    )(eps, acc, deg, feat, W1, b1.reshape(1, d_hid), W2, b2.reshape(1, d_out))
    return out
